# baseline (speedup 1.0000x reference)
import functools

import jax
import jax.numpy as jnp
from jax.experimental import pallas as pl
from jax.experimental.pallas import tpu as pltpu

_B = 64
_INPUT = 2048
_MEM = 8192
_CELL = 128
_HEADS = 16
_K = 8
_MB = 8192
_NMB = _MEM // _MB

_NEG = -3.0e38


def _proj_kernel(x_ref, wq_ref, bq_ref, wv_ref, bv_ref, wg_ref, bg_ref,
                 q_ref, upd_ref):
    x = x_ref[...]
    q_ref[...] = jax.lax.dot_general(
        x, wq_ref[...], (((1,), (0,)), ((), ())),
        preferred_element_type=jnp.float32) + bq_ref[...]
    wv = jax.lax.dot_general(
        x, wv_ref[...], (((1,), (0,)), ((), ())),
        preferred_element_type=jnp.float32) + bv_ref[...]
    g_full = jax.lax.dot_general(
        x, wg_ref[...], (((1,), (0,)), ((), ())),
        preferred_element_type=jnp.float32)
    gate = jax.nn.sigmoid(g_full[:, 0:1] + bg_ref[0, 0])
    upd_ref[...] = gate * wv


def _main_kernel(q_ref, mem_ref, rv_ref, newmem_ref, topi_ref, sem):
    b = pl.program_id(0)
    copy = pltpu.make_async_copy(mem_ref, newmem_ref.at[pl.ds(b, 1)], sem)
    copy.start()
    mem = mem_ref[0]
    q = q_ref[0]
    work = jax.lax.dot_general(
        q, mem, (((1,), (1,)), ((), ())),
        preferred_element_type=jnp.float32)
    col = jax.lax.broadcasted_iota(jnp.int32, (_HEADS, _MEM), 1)
    topv = []
    topi = []
    for _ in range(_K):
        mval = jnp.max(work, axis=1, keepdims=True)
        idx = jnp.min(jnp.where(work == mval, col, _MEM),
                      axis=1, keepdims=True)
        topv.append(mval)
        topi.append(idx)
        work = jnp.where(col == idx, _NEG, work)
    vmax = topv[0]
    expv = [jnp.exp(v - vmax) for v in topv]
    denom = functools.reduce(jnp.add, expv)
    wdense = jnp.zeros((_HEADS, _MEM), jnp.float32)
    for k in range(_K):
        wdense = jnp.where(col == topi[k], expv[k] / denom, wdense)
    rv_ref[0] = jax.lax.dot_general(
        wdense, mem, (((1,), (0,)), ((), ())),
        preferred_element_type=jnp.float32)
    topi_ref[0] = jnp.concatenate(topi, axis=1)
    copy.wait()


def _scatter_kernel(pos_ref, upd_ref, row_ref, out_ref):
    b = pl.program_id(0)
    r = pos_ref[b] % 8
    rows = jax.lax.broadcasted_iota(jnp.int32, (8, 1), 0)
    out_ref[0] = row_ref[0] + jnp.where(rows == r, upd_ref[0, 0], 0.0)


def kernel(x, memory, Wq, bq, Wv, bv, Wg, bg):
    f32 = jnp.float32
    wg_pad = jnp.pad(Wg, ((0, 0), (0, 127)))
    queries, upd = pl.pallas_call(
        _proj_kernel,
        out_shape=(
            jax.ShapeDtypeStruct((_B, _HEADS * _CELL), f32),
            jax.ShapeDtypeStruct((_B, _CELL), f32),
        ),
    )(x, Wq, bq.reshape(1, -1), Wv, bv.reshape(1, -1), wg_pad,
      bg.reshape(1, 1))
    queries = queries.reshape(_B, _HEADS, _CELL)

    read_vectors, new_memory, topi = pl.pallas_call(
        _main_kernel,
        grid=(_B,),
        in_specs=[
            pl.BlockSpec((1, _HEADS, _CELL), lambda b: (b, 0, 0)),
            pl.BlockSpec((1, _MEM, _CELL), lambda b: (b, 0, 0)),
        ],
        out_specs=[
            pl.BlockSpec((1, _HEADS, _CELL), lambda b: (b, 0, 0)),
            pl.BlockSpec(memory_space=pl.ANY),
            pl.BlockSpec((1, _HEADS, _K), lambda b: (b, 0, 0)),
        ],
        out_shape=(
            jax.ShapeDtypeStruct((_B, _HEADS, _CELL), f32),
            jax.ShapeDtypeStruct((_B, _MEM, _CELL), f32),
            jax.ShapeDtypeStruct((_B, _HEADS, _K), jnp.int32),
        ),
        scratch_shapes=[pltpu.SemaphoreType.DMA],
    )(queries, memory)

    pos = topi[:, 0, 0]
    new_memory = pl.pallas_call(
        _scatter_kernel,
        grid_spec=pltpu.PrefetchScalarGridSpec(
            num_scalar_prefetch=1,
            grid=(_B,),
            in_specs=[
                pl.BlockSpec((1, 1, _CELL), lambda b, pos_ref: (b, 0, 0)),
                pl.BlockSpec((1, 8, _CELL),
                             lambda b, pos_ref: (b, pos_ref[b] // 8, 0)),
            ],
            out_specs=pl.BlockSpec((1, 8, _CELL),
                                   lambda b, pos_ref: (b, pos_ref[b] // 8, 0)),
        ),
        out_shape=jax.ShapeDtypeStruct((_B, _MEM, _CELL), f32),
        input_output_aliases={2: 0},
    )(pos, upd.reshape(_B, 1, _CELL), new_memory)

    return read_vectors, new_memory


# speedup vs baseline: 1.2204x; 1.2204x over previous
import functools

import jax
import jax.numpy as jnp
from jax.experimental import pallas as pl
from jax.experimental.pallas import tpu as pltpu

_B = 64
_INPUT = 2048
_MEM = 8192
_CELL = 128
_HEADS = 16
_K = 8
_MB = 8192
_NMB = _MEM // _MB

_NEG = -3.0e38


def _proj_kernel(x_ref, wq_ref, bq_ref, wv_ref, bv_ref, wg_ref, bg_ref,
                 q_ref, upd_ref):
    x = x_ref[...]
    q_ref[...] = jax.lax.dot_general(
        x, wq_ref[...], (((1,), (0,)), ((), ())),
        preferred_element_type=jnp.float32) + bq_ref[...]
    wv = jax.lax.dot_general(
        x, wv_ref[...], (((1,), (0,)), ((), ())),
        preferred_element_type=jnp.float32) + bv_ref[...]
    g_full = jax.lax.dot_general(
        x, wg_ref[...], (((1,), (0,)), ((), ())),
        preferred_element_type=jnp.float32)
    gate = jax.nn.sigmoid(g_full[:, 0:1] + bg_ref[0, 0])
    upd_ref[...] = gate * wv


def _main_kernel(q_ref, mem_ref, rv_ref, newmem_ref, topi_ref, sem):
    b = pl.program_id(0)
    copy = pltpu.make_async_copy(mem_ref, newmem_ref.at[pl.ds(b, 1)], sem)
    copy.start()
    mem = mem_ref[0]
    q = q_ref[0]
    scores = jax.lax.dot_general(
        q, mem, (((1,), (1,)), ((), ())),
        preferred_element_type=jnp.float32)
    work = scores
    topv = []
    for _ in range(_K):
        mval = jnp.max(work, axis=1, keepdims=True)
        topv.append(mval)
        work = jnp.where(work == mval, _NEG, work)
    vmax = topv[0]
    denom = functools.reduce(jnp.add, [jnp.exp(v - vmax) for v in topv])
    inv = 1.0 / denom
    wdense = jnp.where(scores >= topv[_K - 1],
                       jnp.exp(scores - vmax) * inv, 0.0)
    rv_ref[0] = jax.lax.dot_general(
        wdense, mem, (((1,), (0,)), ((), ())),
        preferred_element_type=jnp.float32)
    s0 = scores[0:1]
    col0 = jax.lax.broadcasted_iota(jnp.int32, (1, _MEM), 1)
    pos = jnp.min(jnp.where(s0 == vmax[0:1], col0, _MEM),
                  axis=1, keepdims=True)
    topi_ref[0] = jnp.broadcast_to(pos, (_HEADS, _K)).astype(jnp.int32)
    copy.wait()


def _scatter_kernel(pos_ref, upd_ref, row_ref, out_ref):
    b = pl.program_id(0)
    r = pos_ref[b] % 8
    rows = jax.lax.broadcasted_iota(jnp.int32, (8, 1), 0)
    out_ref[0] = row_ref[0] + jnp.where(rows == r, upd_ref[0, 0], 0.0)


def kernel(x, memory, Wq, bq, Wv, bv, Wg, bg):
    f32 = jnp.float32
    wg_pad = jnp.pad(Wg, ((0, 0), (0, 127)))
    queries, upd = pl.pallas_call(
        _proj_kernel,
        out_shape=(
            jax.ShapeDtypeStruct((_B, _HEADS * _CELL), f32),
            jax.ShapeDtypeStruct((_B, _CELL), f32),
        ),
    )(x, Wq, bq.reshape(1, -1), Wv, bv.reshape(1, -1), wg_pad,
      bg.reshape(1, 1))
    queries = queries.reshape(_B, _HEADS, _CELL)

    read_vectors, new_memory, topi = pl.pallas_call(
        _main_kernel,
        grid=(_B,),
        in_specs=[
            pl.BlockSpec((1, _HEADS, _CELL), lambda b: (b, 0, 0)),
            pl.BlockSpec((1, _MEM, _CELL), lambda b: (b, 0, 0)),
        ],
        out_specs=[
            pl.BlockSpec((1, _HEADS, _CELL), lambda b: (b, 0, 0)),
            pl.BlockSpec(memory_space=pl.ANY),
            pl.BlockSpec((1, _HEADS, _K), lambda b: (b, 0, 0)),
        ],
        out_shape=(
            jax.ShapeDtypeStruct((_B, _HEADS, _CELL), f32),
            jax.ShapeDtypeStruct((_B, _MEM, _CELL), f32),
            jax.ShapeDtypeStruct((_B, _HEADS, _K), jnp.int32),
        ),
        scratch_shapes=[pltpu.SemaphoreType.DMA],
    )(queries, memory)

    pos = topi[:, 0, 0]
    new_memory = pl.pallas_call(
        _scatter_kernel,
        grid_spec=pltpu.PrefetchScalarGridSpec(
            num_scalar_prefetch=1,
            grid=(_B,),
            in_specs=[
                pl.BlockSpec((1, 1, _CELL), lambda b, pos_ref: (b, 0, 0)),
                pl.BlockSpec((1, 8, _CELL),
                             lambda b, pos_ref: (b, pos_ref[b] // 8, 0)),
            ],
            out_specs=pl.BlockSpec((1, 8, _CELL),
                                   lambda b, pos_ref: (b, pos_ref[b] // 8, 0)),
        ),
        out_shape=jax.ShapeDtypeStruct((_B, _MEM, _CELL), f32),
        input_output_aliases={2: 0},
    )(pos, upd.reshape(_B, 1, _CELL), new_memory)

    return read_vectors, new_memory


# speedup vs baseline: 1.3319x; 1.0914x over previous
import functools

import jax
import jax.numpy as jnp
from jax.experimental import pallas as pl
from jax.experimental.pallas import tpu as pltpu

_B = 64
_MEM = 8192
_CELL = 128
_HEADS = 16
_K = 8

_NEG = -3.0e38


def _proj_kernel(x_ref, wq_ref, bq_ref, wv_ref, bv_ref, wg_ref, bg_ref,
                 q_ref, upd_ref):
    x = x_ref[...]
    q_ref[...] = jax.lax.dot_general(
        x, wq_ref[...], (((1,), (0,)), ((), ())),
        preferred_element_type=jnp.float32) + bq_ref[...]
    wv = jax.lax.dot_general(
        x, wv_ref[...], (((1,), (0,)), ((), ())),
        preferred_element_type=jnp.float32) + bv_ref[...]
    g_full = jax.lax.dot_general(
        x, wg_ref[...], (((1,), (0,)), ((), ())),
        preferred_element_type=jnp.float32)
    gate = jax.nn.sigmoid(g_full[:, 0:1] + bg_ref[0, 0])
    upd_ref[...] = gate * wv


def _main_kernel(q_ref, mem_ref, upd_ref, rv_ref, newmem_ref,
                 row_scr, sem, sem2):
    b = pl.program_id(0)
    copy = pltpu.make_async_copy(mem_ref, newmem_ref.at[pl.ds(b, 1)], sem)
    copy.start()
    mem = mem_ref[0]
    q = q_ref[0]
    scores = jax.lax.dot_general(
        q, mem, (((1,), (1,)), ((), ())),
        preferred_element_type=jnp.float32)
    work = scores
    topv = []
    for _ in range(_K):
        mval = jnp.max(work, axis=1, keepdims=True)
        topv.append(mval)
        work = jnp.where(work == mval, _NEG, work)
    vmax = topv[0]
    denom = functools.reduce(jnp.add, [jnp.exp(v - vmax) for v in topv])
    inv = 1.0 / denom
    wdense = jnp.where(scores >= topv[_K - 1],
                       jnp.exp(scores - vmax) * inv, 0.0)
    rv_ref[0] = jax.lax.dot_general(
        wdense, mem, (((1,), (0,)), ((), ())),
        preferred_element_type=jnp.float32)
    s0 = scores[0:1]
    col0 = jax.lax.broadcasted_iota(jnp.int32, (1, _MEM), 1)
    pos = jnp.min(jnp.where(s0 == vmax[0:1], col0, _MEM))
    par = b % 2
    row_scr[pl.ds(par, 1), :] = mem_ref[0, pl.ds(pos, 1), :] + upd_ref[0]
    copy.wait()

    @pl.when(b > 0)
    def _drain_prev():
        pltpu.make_async_copy(
            row_scr.at[pl.ds(1 - par, 1)], newmem_ref.at[b, pl.ds(0, 1)],
            sem2).wait()

    patch = pltpu.make_async_copy(
        row_scr.at[pl.ds(par, 1)], newmem_ref.at[b, pl.ds(pos, 1)], sem2)
    patch.start()

    @pl.when(b == _B - 1)
    def _drain_last():
        patch.wait()


def kernel(x, memory, Wq, bq, Wv, bv, Wg, bg):
    f32 = jnp.float32
    wg_pad = jnp.pad(Wg, ((0, 0), (0, 127)))
    queries, upd = pl.pallas_call(
        _proj_kernel,
        out_shape=(
            jax.ShapeDtypeStruct((_B, _HEADS * _CELL), f32),
            jax.ShapeDtypeStruct((_B, _CELL), f32),
        ),
    )(x, Wq, bq.reshape(1, -1), Wv, bv.reshape(1, -1), wg_pad,
      bg.reshape(1, 1))
    queries = queries.reshape(_B, _HEADS, _CELL)

    read_vectors, new_memory = pl.pallas_call(
        _main_kernel,
        grid=(_B,),
        in_specs=[
            pl.BlockSpec((1, _HEADS, _CELL), lambda b: (b, 0, 0)),
            pl.BlockSpec((1, _MEM, _CELL), lambda b: (b, 0, 0)),
            pl.BlockSpec((1, 1, _CELL), lambda b: (b, 0, 0)),
        ],
        out_specs=[
            pl.BlockSpec((1, _HEADS, _CELL), lambda b: (b, 0, 0)),
            pl.BlockSpec(memory_space=pl.ANY),
        ],
        out_shape=(
            jax.ShapeDtypeStruct((_B, _HEADS, _CELL), f32),
            jax.ShapeDtypeStruct((_B, _MEM, _CELL), f32),
        ),
        scratch_shapes=[
            pltpu.VMEM((2, _CELL), f32),
            pltpu.SemaphoreType.DMA,
            pltpu.SemaphoreType.DMA,
        ],
    )(queries, memory, upd.reshape(_B, 1, _CELL))

    return read_vectors, new_memory
